# baseline (speedup 1.0000x reference)
import functools

import jax
import jax.numpy as jnp
from jax import lax
from jax.experimental import pallas as pl
from jax.experimental.pallas import tpu as pltpu
from jax.experimental.pallas import tpu_sc as plsc

_GATHER = 128
_BC = 256
_NG = _BC // _GATHER
_TPAD = 261


@functools.lru_cache(maxsize=None)
def _build(batch: int, hist: int, vocab: int, dim: int):
    info = plsc.get_sparse_core_info()
    nw = info.num_cores * info.num_subcores
    blocks_per_t = batch // _BC
    nchunk = hist * blocks_per_t // nw
    npair = nchunk // 2
    ntc = dim // 8
    ntb = batch // 128
    mesh = plsc.VectorSubcoreMesh(core_axis_name="c", subcore_axis_name="s")

    @functools.partial(
        pl.kernel,
        mesh=mesh,
        compiler_params=pltpu.CompilerParams(
            use_tc_tiling_on_sc=False, needs_layout_passes=False),
        out_type=jax.ShapeDtypeStruct((hist, ntc, ntb, 8, 128), jnp.float32),
        scratch_types=[
            pltpu.VMEM((2, _BC), jnp.int32),
            pltpu.VMEM((_BC, dim), jnp.float32),
            pltpu.VMEM((_BC, dim), jnp.float32),
            pltpu.VMEM((ntc, 8, _TPAD), jnp.float32),
            pltpu.VMEM((ntc, 8, _TPAD), jnp.float32),
            pltpu.SemaphoreType.DMA,
            pltpu.SemaphoreType.DMA,
            pltpu.SemaphoreType.DMA,
            pltpu.SemaphoreType.DMA,
            pltpu.SemaphoreType.DMA,
            pltpu.SemaphoreType.DMA,
        ],
    )
    def emb(idxt_hbm, table_hbm, out_hbm, idx_v, rv0, rv1, rt0, rt1,
            si0, si1, sgA, sgB, sw0, sw1):
        wid = lax.axis_index("s") * info.num_cores + lax.axis_index("c")
        cid0 = wid * nchunk

        def t_of(c):
            return (cid0 + c) // blocks_per_t

        def b0_of(c):
            return ((cid0 + c) % blocks_per_t) * _BC

        def idx_start(c, b, sem):
            pltpu.async_copy(
                idxt_hbm.at[t_of(c), pl.ds(b0_of(c), _BC)], idx_v.at[b], sem)

        def idx_wait(b, sem):
            pltpu.make_async_copy(
                idxt_hbm.at[0, pl.ds(0, _BC)], idx_v.at[b], sem).wait()

        def gather_start(b, rv, sem):
            for g in range(_NG):
                pltpu.async_copy(
                    table_hbm.at[idx_v.at[b, pl.ds(g * _GATHER, _GATHER)]],
                    rv.at[pl.ds(g * _GATHER, _GATHER)],
                    sem,
                )

        def gather_wait(b, rv, sem):
            for g in range(_NG):
                pltpu.make_async_copy(
                    table_hbm.at[idx_v.at[b, pl.ds(g * _GATHER, _GATHER)]],
                    rv.at[pl.ds(g * _GATHER, _GATHER)],
                    sem,
                ).wait()

        def wo_start(c, rt, sem):
            for bj in range(_BC // 128):
                pltpu.async_copy(
                    rt.at[:, :, pl.ds(bj * 128, 128)],
                    out_hbm.at[t_of(c), :, b0_of(c) // 128 + bj],
                    sem,
                )

        def wo_wait(rt, sem):
            for bj in range(_BC // 128):
                pltpu.make_async_copy(
                    rt.at[:, :, pl.ds(bj * 128, 128)],
                    out_hbm.at[0, :, bj],
                    sem,
                ).wait()

        def transpose(rv, rt):
            lanes = lax.iota(jnp.int32, 16)
            cvecs = [jnp.full((16,), c, jnp.int32) for c in range(dim)]

            def tbody(grp, carry):
                brange = lanes + grp * 16
                for c in range(dim):
                    val = plsc.load_gather(rv, [brange, cvecs[c]])
                    rt[c // 8, c % 8, pl.ds(grp * 16, 16)] = val
                return carry

            lax.fori_loop(0, _BC // 16, tbody, None)

        idx_start(0, 0, si0)
        idx_start(1, 1, si1)
        idx_wait(0, si0)
        gather_start(0, rv0, sgA)

        def body(i, carry):
            j0 = i * 2

            idx_wait(1, si1)
            gather_wait(0, rv0, sgA)
            gather_start(1, rv1, sgB)

            @pl.when(j0 + 2 < nchunk)
            def _():
                idx_start(j0 + 2, 0, si0)

            @pl.when(i > 0)
            def _():
                wo_wait(rt0, sw0)

            transpose(rv0, rt0)
            wo_start(j0, rt0, sw0)

            gather_wait(1, rv1, sgB)

            @pl.when(j0 + 3 < nchunk)
            def _():
                idx_start(j0 + 3, 1, si1)

            @pl.when(j0 + 2 < nchunk)
            def _():
                idx_wait(0, si0)
                gather_start(0, rv0, sgA)

            @pl.when(i > 0)
            def _():
                wo_wait(rt1, sw1)

            transpose(rv1, rt1)
            wo_start(j0 + 1, rt1, sw1)
            return carry

        lax.fori_loop(0, npair, body, None)

        wo_wait(rt0, sw0)
        wo_wait(rt1, sw1)

    return emb


def kernel(inputs, table):
    batch, hist = inputs.shape
    vocab, dim = table.shape
    idx_t = inputs.T.astype(jnp.int32)
    o5 = _build(batch, hist, vocab, dim)(idx_t, table)
    return o5.transpose(2, 4, 0, 1, 3).reshape(batch, hist, dim)


# speedup vs baseline: 1.8585x; 1.8585x over previous
import functools

import jax
import jax.numpy as jnp
from jax import lax
from jax.experimental import pallas as pl
from jax.experimental.pallas import tpu as pltpu
from jax.experimental.pallas import tpu_sc as plsc

_GATHER = 128
_BC = 256
_NG = _BC // _GATHER
_TPAD = 261


@functools.lru_cache(maxsize=None)
def _build(batch: int, hist: int, vocab: int, dim: int):
    info = plsc.get_sparse_core_info()
    nw = info.num_cores * info.num_subcores
    blocks_per_t = batch // _BC
    nchunk = hist * blocks_per_t // nw
    npair = nchunk // 2
    ntc = dim // 8
    ntb = batch // 128
    mesh = plsc.VectorSubcoreMesh(core_axis_name="c", subcore_axis_name="s")

    @functools.partial(
        pl.kernel,
        mesh=mesh,
        compiler_params=pltpu.CompilerParams(
            use_tc_tiling_on_sc=False, needs_layout_passes=False),
        out_type=jax.ShapeDtypeStruct((hist, ntc, ntb, 8, 128), jnp.float32),
        scratch_types=[
            pltpu.VMEM((2, _BC), jnp.int32),
            pltpu.VMEM((_BC, dim), jnp.float32),
            pltpu.VMEM((_BC, dim), jnp.float32),
            pltpu.VMEM((ntc, 8, _TPAD), jnp.float32),
            pltpu.VMEM((ntc, 8, _TPAD), jnp.float32),
            pltpu.SemaphoreType.DMA,
            pltpu.SemaphoreType.DMA,
            pltpu.SemaphoreType.DMA,
            pltpu.SemaphoreType.DMA,
            pltpu.SemaphoreType.DMA,
            pltpu.SemaphoreType.DMA,
        ],
    )
    def emb(idxt_hbm, table_hbm, out_hbm, idx_v, rv0, rv1, rt0, rt1,
            si0, si1, sgA, sgB, sw0, sw1):
        wid = lax.axis_index("s") * info.num_cores + lax.axis_index("c")
        cid0 = wid * nchunk

        def t_of(c):
            return (cid0 + c) // blocks_per_t

        def b0_of(c):
            return ((cid0 + c) % blocks_per_t) * _BC

        def idx_start(c, b, sem):
            pltpu.async_copy(
                idxt_hbm.at[t_of(c), pl.ds(b0_of(c), _BC)], idx_v.at[b], sem)

        def idx_wait(b, sem):
            pltpu.make_async_copy(
                idxt_hbm.at[0, pl.ds(0, _BC)], idx_v.at[b], sem).wait()

        def gather_start(b, rv, sem):
            for g in range(_NG):
                pltpu.async_copy(
                    table_hbm.at[idx_v.at[b, pl.ds(g * _GATHER, _GATHER)]],
                    rv.at[pl.ds(g * _GATHER, _GATHER)],
                    sem,
                )

        def gather_wait(b, rv, sem):
            for g in range(_NG):
                pltpu.make_async_copy(
                    table_hbm.at[idx_v.at[b, pl.ds(g * _GATHER, _GATHER)]],
                    rv.at[pl.ds(g * _GATHER, _GATHER)],
                    sem,
                ).wait()

        def wo_start(c, rt, sem):
            for bj in range(_BC // 128):
                pltpu.async_copy(
                    rt.at[:, :, pl.ds(bj * 128, 128)],
                    out_hbm.at[t_of(c), :, b0_of(c) // 128 + bj],
                    sem,
                )

        def wo_wait(rt, sem):
            for bj in range(_BC // 128):
                pltpu.make_async_copy(
                    rt.at[:, :, pl.ds(bj * 128, 128)],
                    out_hbm.at[0, :, bj],
                    sem,
                ).wait()

        def transpose(rv, rt):
            lanes = lax.iota(jnp.int32, 16)
            civecs = [(lanes + g * 16) // 8 for g in range(dim // 16)]
            clvecs = [(lanes + g * 16) % 8 for g in range(dim // 16)]

            def tbody(bb, carry):
                for u in range(4):
                    b = bb * 4 + u
                    bvec = jnp.full((16,), b, jnp.int32)
                    for g in range(dim // 16):
                        val = rv[b, pl.ds(g * 16, 16)]
                        plsc.store_scatter(
                            rt, [civecs[g], clvecs[g], bvec], val)
                return carry

            lax.fori_loop(0, _BC // 4, tbody, None)

        idx_start(0, 0, si0)
        idx_start(1, 1, si1)
        idx_wait(0, si0)
        gather_start(0, rv0, sgA)

        def body(i, carry):
            j0 = i * 2

            idx_wait(1, si1)
            gather_wait(0, rv0, sgA)
            gather_start(1, rv1, sgB)

            @pl.when(j0 + 2 < nchunk)
            def _():
                idx_start(j0 + 2, 0, si0)

            @pl.when(i > 0)
            def _():
                wo_wait(rt0, sw0)

            transpose(rv0, rt0)
            wo_start(j0, rt0, sw0)

            gather_wait(1, rv1, sgB)

            @pl.when(j0 + 3 < nchunk)
            def _():
                idx_start(j0 + 3, 1, si1)

            @pl.when(j0 + 2 < nchunk)
            def _():
                idx_wait(0, si0)
                gather_start(0, rv0, sgA)

            @pl.when(i > 0)
            def _():
                wo_wait(rt1, sw1)

            transpose(rv1, rt1)
            wo_start(j0 + 1, rt1, sw1)
            return carry

        lax.fori_loop(0, npair, body, None)

        wo_wait(rt0, sw0)
        wo_wait(rt1, sw1)

    return emb


def kernel(inputs, table):
    batch, hist = inputs.shape
    vocab, dim = table.shape
    idx_t = inputs.T.astype(jnp.int32)
    o5 = _build(batch, hist, vocab, dim)(idx_t, table)
    return o5.transpose(2, 4, 0, 1, 3).reshape(batch, hist, dim)
